# baseline (device time: 122536 ns/iter reference)
import functools

import jax
import jax.numpy as jnp
from jax import lax
from jax.experimental import pallas as pl
from jax.experimental.pallas import tpu as pltpu

N_DEV = 4
B = 2
S = 512
H = 8
DH = 64
HD = H * DH
D = 768
BLK = 64


def kernel(x, Wq, K_ext, V_ext, Wo):
    xb = x.astype(jnp.bfloat16)
    wq = Wq.astype(jnp.bfloat16)
    k2 = K_ext.reshape(B, S, HD).astype(jnp.bfloat16)
    v2 = V_ext.reshape(B, S, HD).astype(jnp.bfloat16)
    wo = Wo.astype(jnp.bfloat16)

    def body(x_ref, wq_ref, k_ref, v_ref, wo_ref, out_ref,
             kg_ref, vg_ref, scores_ref, ctx_ref,
             local_sems, ksend_sems, krecv_sems, vsend_sems, vrecv_sems):
        my = lax.axis_index("i")
        left = lax.rem(my + N_DEV - 1, N_DEV)
        right = lax.rem(my + 1, N_DEV)

        barrier_sem = pltpu.get_barrier_semaphore()
        for nbr in [left, right]:
            pl.semaphore_signal(
                barrier_sem, inc=1,
                device_id=(nbr,), device_id_type=pl.DeviceIdType.MESH,
            )
        pl.semaphore_wait(barrier_sem, 2)

        kcopy = pltpu.make_async_copy(k_ref, kg_ref.at[my], local_sems.at[0])
        vcopy = pltpu.make_async_copy(v_ref, vg_ref.at[my], local_sems.at[1])
        kcopy.start()
        vcopy.start()
        kcopy.wait()
        vcopy.wait()

        for h in range(1, N_DEV):
            o_send = lax.rem(my - h + 1 + N_DEV, N_DEV)
            o_recv = lax.rem(my - h + N_DEV, N_DEV)

            k_send = pltpu.make_async_remote_copy(
                src_ref=kg_ref.at[o_send], dst_ref=kg_ref.at[o_send],
                send_sem=ksend_sems.at[o_send], recv_sem=krecv_sems.at[o_send],
                device_id=(right,), device_id_type=pl.DeviceIdType.MESH,
            )
            v_send = pltpu.make_async_remote_copy(
                src_ref=vg_ref.at[o_send], dst_ref=vg_ref.at[o_send],
                send_sem=vsend_sems.at[o_send], recv_sem=vrecv_sems.at[o_send],
                device_id=(right,), device_id_type=pl.DeviceIdType.MESH,
            )
            k_send.start()
            v_send.start()

            k_recv = pltpu.make_async_remote_copy(
                src_ref=kg_ref.at[o_recv], dst_ref=kg_ref.at[o_recv],
                send_sem=ksend_sems.at[o_recv], recv_sem=krecv_sems.at[o_recv],
                device_id=(right,), device_id_type=pl.DeviceIdType.MESH,
            )
            v_recv = pltpu.make_async_remote_copy(
                src_ref=vg_ref.at[o_recv], dst_ref=vg_ref.at[o_recv],
                send_sem=vsend_sems.at[o_recv], recv_sem=vrecv_sems.at[o_recv],
                device_id=(right,), device_id_type=pl.DeviceIdType.MESH,
            )
            k_send.wait_send()
            v_send.wait_send()
            k_recv.wait_recv()
            v_recv.wait_recv()

        rows = jax.lax.broadcasted_iota(jnp.int32, (S, N_DEV * S), 0)
        cols = jax.lax.broadcasted_iota(jnp.int32, (S, N_DEV * S), 1)
        qb = (my * S + rows) // BLK
        kb = cols // BLK
        mask = kb <= qb

        for b in range(B):
            q2 = lax.dot(
                x_ref[b], wq_ref[...], preferred_element_type=jnp.float32
            ).astype(jnp.bfloat16)
            for h in range(H):
                q_bh = q2[:, h * DH:(h + 1) * DH]
                for s in range(N_DEV):
                    k_sb = kg_ref[s, b, :, h * DH:(h + 1) * DH]
                    scores_ref[:, s * S:(s + 1) * S] = lax.dot_general(
                        q_bh, k_sb,
                        dimension_numbers=(((1,), (1,)), ((), ())),
                        preferred_element_type=jnp.float32,
                    )
                scores = scores_ref[...] * 0.125
                scores = jnp.where(mask, scores, -1e9)
                smax = jnp.max(scores, axis=-1, keepdims=True)
                w = jnp.exp(scores - smax)
                w = w / jnp.sum(w, axis=-1, keepdims=True)
                wb = w.astype(jnp.bfloat16)
                ctx = jnp.zeros((S, DH), jnp.float32)
                for s in range(N_DEV):
                    v_sb = vg_ref[s, b, :, h * DH:(h + 1) * DH]
                    ctx = ctx + lax.dot(
                        wb[:, s * S:(s + 1) * S], v_sb,
                        preferred_element_type=jnp.float32,
                    )
                ctx_ref[:, h * DH:(h + 1) * DH] = ctx
            out_ref[b] = lax.dot(
                ctx_ref[...].astype(jnp.bfloat16), wo_ref[...],
                preferred_element_type=jnp.float32,
            )

        @functools.partial(
            pl.run_scoped, second_barrier=pltpu.SemaphoreType.REGULAR
        )
        def _(second_barrier):
            for nbr in [left, right]:
                pl.semaphore_signal(
                    second_barrier, inc=1,
                    device_id=(nbr,), device_id_type=pl.DeviceIdType.MESH,
                )
            pl.semaphore_wait(second_barrier, 2)

    return pl.pallas_call(
        body,
        out_shape=jax.ShapeDtypeStruct((B, S, D), jnp.float32),
        in_specs=[pl.BlockSpec(memory_space=pltpu.VMEM)] * 5,
        out_specs=pl.BlockSpec(memory_space=pltpu.VMEM),
        scratch_shapes=[
            pltpu.VMEM((N_DEV, B, S, HD), jnp.bfloat16),
            pltpu.VMEM((N_DEV, B, S, HD), jnp.bfloat16),
            pltpu.VMEM((S, N_DEV * S), jnp.float32),
            pltpu.VMEM((S, HD), jnp.float32),
            pltpu.SemaphoreType.DMA((2,)),
            pltpu.SemaphoreType.DMA((N_DEV,)),
            pltpu.SemaphoreType.DMA((N_DEV,)),
            pltpu.SemaphoreType.DMA((N_DEV,)),
            pltpu.SemaphoreType.DMA((N_DEV,)),
        ],
        compiler_params=pltpu.CompilerParams(collective_id=0),
    )(xb, wq, k2, v2, wo)


# device time: 100745 ns/iter; 1.2163x vs baseline; 1.2163x over previous
import functools

import jax
import jax.numpy as jnp
from jax import lax
from jax.experimental import pallas as pl
from jax.experimental.pallas import tpu as pltpu

N_DEV = 4
B = 2
S = 512
H = 8
DH = 64
HD = H * DH
D = 768
BLK = 64


def kernel(x, Wq, K_ext, V_ext, Wo):
    xb = x.astype(jnp.bfloat16)
    wq = Wq.astype(jnp.bfloat16)
    k2 = K_ext.reshape(B, S, HD).astype(jnp.bfloat16)
    v2 = V_ext.reshape(B, S, HD).astype(jnp.bfloat16)
    wo = Wo.astype(jnp.bfloat16)

    def body(x_ref, wq_ref, k_ref, v_ref, wo_ref, out_ref,
             kg_ref, vg_ref, ctx_ref,
             ksend_sems, krecv_sems, vsend_sems, vrecv_sems):
        my = lax.axis_index("i")
        left = lax.rem(my + N_DEV - 1, N_DEV)
        right = lax.rem(my + 1, N_DEV)

        barrier_sem = pltpu.get_barrier_semaphore()
        for nbr in [left, right]:
            pl.semaphore_signal(
                barrier_sem, inc=1,
                device_id=(nbr,), device_id_type=pl.DeviceIdType.MESH,
            )
        pl.semaphore_wait(barrier_sem, 2)

        def send_hop(h):
            ksrc = k_ref if h == 1 else kg_ref.at[h - 2]
            vsrc = v_ref if h == 1 else vg_ref.at[h - 2]
            kr = pltpu.make_async_remote_copy(
                src_ref=ksrc, dst_ref=kg_ref.at[h - 1],
                send_sem=ksend_sems.at[h - 1], recv_sem=krecv_sems.at[h - 1],
                device_id=(right,), device_id_type=pl.DeviceIdType.MESH,
            )
            vr = pltpu.make_async_remote_copy(
                src_ref=vsrc, dst_ref=vg_ref.at[h - 1],
                send_sem=vsend_sems.at[h - 1], recv_sem=vrecv_sems.at[h - 1],
                device_id=(right,), device_id_type=pl.DeviceIdType.MESH,
            )
            kr.start()
            vr.start()
            return kr, vr

        def wait_hop_recv(h):
            for g_ref, sems in ((kg_ref, krecv_sems), (vg_ref, vrecv_sems)):
                pltpu.make_async_remote_copy(
                    src_ref=g_ref.at[h - 1], dst_ref=g_ref.at[h - 1],
                    send_sem=sems.at[h - 1], recv_sem=sems.at[h - 1],
                    device_id=(right,), device_id_type=pl.DeviceIdType.MESH,
                ).wait_recv()

        sent = list(send_hop(1))

        q2 = [
            (lax.dot(x_ref[b], wq_ref[...],
                     preferred_element_type=jnp.float32) * 0.125
             ).astype(jnp.bfloat16)
            for b in range(B)
        ]

        rows = jax.lax.broadcasted_iota(jnp.int32, (S, S), 0)
        cols = jax.lax.broadcasted_iota(jnp.int32, (S, S), 1)
        qb = (my * S + rows) // BLK

        l_acc = [[None] * H for _ in range(B)]
        o_acc = [[None] * H for _ in range(B)]

        def accumulate(origin, kc_ref, vc_ref):
            mask = (origin * S + cols) // BLK <= qb
            for b in range(B):
                for h in range(H):
                    q_bh = q2[b][:, h * DH:(h + 1) * DH]
                    k_bh = kc_ref[b, :, h * DH:(h + 1) * DH]
                    s = lax.dot_general(
                        q_bh, k_bh,
                        dimension_numbers=(((1,), (1,)), ((), ())),
                        preferred_element_type=jnp.float32,
                    )
                    p = jnp.where(mask, jnp.exp(s), 0.0)
                    lsum = jnp.sum(p, axis=-1, keepdims=True)
                    pv = lax.dot(
                        p.astype(jnp.bfloat16),
                        vc_ref[b, :, h * DH:(h + 1) * DH],
                        preferred_element_type=jnp.float32,
                    )
                    if l_acc[b][h] is None:
                        l_acc[b][h] = lsum
                        o_acc[b][h] = pv
                    else:
                        l_acc[b][h] = l_acc[b][h] + lsum
                        o_acc[b][h] = o_acc[b][h] + pv

        accumulate(my, k_ref, v_ref)

        for h in range(1, N_DEV):
            wait_hop_recv(h)
            if h < N_DEV - 1:
                sent += list(send_hop(h + 1))
            origin = lax.rem(my - h + N_DEV, N_DEV)
            accumulate(origin, kg_ref.at[h - 1], vg_ref.at[h - 1])

        for b in range(B):
            for h in range(H):
                ctx_ref[:, h * DH:(h + 1) * DH] = o_acc[b][h] / l_acc[b][h]
            out_ref[b] = lax.dot(
                ctx_ref[...].astype(jnp.bfloat16), wo_ref[...],
                preferred_element_type=jnp.float32,
            )

        for r in sent:
            r.wait_send()

        @functools.partial(
            pl.run_scoped, second_barrier=pltpu.SemaphoreType.REGULAR
        )
        def _(second_barrier):
            for nbr in [left, right]:
                pl.semaphore_signal(
                    second_barrier, inc=1,
                    device_id=(nbr,), device_id_type=pl.DeviceIdType.MESH,
                )
            pl.semaphore_wait(second_barrier, 2)

    return pl.pallas_call(
        body,
        out_shape=jax.ShapeDtypeStruct((B, S, D), jnp.float32),
        in_specs=[pl.BlockSpec(memory_space=pltpu.VMEM)] * 5,
        out_specs=pl.BlockSpec(memory_space=pltpu.VMEM),
        scratch_shapes=[
            pltpu.VMEM((N_DEV - 1, B, S, HD), jnp.bfloat16),
            pltpu.VMEM((N_DEV - 1, B, S, HD), jnp.bfloat16),
            pltpu.VMEM((S, HD), jnp.float32),
            pltpu.SemaphoreType.DMA((N_DEV - 1,)),
            pltpu.SemaphoreType.DMA((N_DEV - 1,)),
            pltpu.SemaphoreType.DMA((N_DEV - 1,)),
            pltpu.SemaphoreType.DMA((N_DEV - 1,)),
        ],
        compiler_params=pltpu.CompilerParams(
            collective_id=0, vmem_limit_bytes=64 * 1024 * 1024
        ),
    )(xb, wq, k2, v2, wo)
